# baseline (speedup 1.0000x reference)
import functools

import jax
import jax.numpy as jnp
from jax import lax
from jax.experimental import pallas as pl
from jax.experimental.pallas import tpu as pltpu
from jax.experimental.pallas import tpu_sc as plsc

NSUB = 16
NG = 2
NS = 2
CH = 128
BM = 1000
EPS = 1e-5


def _make_sc_seg(n, dq, nch, npass):
    td = dq // 16
    assert nch % 4 == 3 and nch >= 7
    zr = 80
    rpt = -(-(-(-n // NSUB)) // zr) * zr
    nzero = rpt // zr
    kq = 2 * npass
    mesh = plsc.VectorSubcoreMesh(core_axis_name="c", subcore_axis_name="s",
                                  num_cores=2, num_subcores=NSUB)

    @functools.partial(
        pl.kernel,
        out_type=jax.ShapeDtypeStruct((kq * n, dq), jnp.float32),
        mesh=mesh,
        scratch_types=[
            pltpu.VMEM((nch, CH), jnp.int32),
            pltpu.VMEM((nch, CH), jnp.int32),
            [pltpu.VMEM((CH * 16,), jnp.float32)] * NG,
            [pltpu.VMEM((CH, dq), jnp.float32)] * NG,
            [pltpu.VMEM((CH, dq), jnp.float32)] * NS,
            pltpu.VMEM((zr, dq), jnp.float32),
            [pltpu.VMEM((zr, dq), jnp.float32)] * 2,
            pltpu.VMEM_SHARED((NSUB * rpt, dq), jnp.float32),
            [pltpu.SemaphoreType.DMA] * NG,
            [pltpu.SemaphoreType.DMA] * NS,
            pltpu.SemaphoreType.DMA,
        ],
        compiler_params=pltpu.CompilerParams(use_tc_tiling_on_sc=False),
    )
    def k(xr_hbm, src_hbm, dst_hbm, w_hbm, out_hbm,
          src_v, dst_v, wbufs, gbufs, mbufs, zbuf, obufs, acc,
          gsems, ssems, wsem):
        c = lax.axis_index("c")
        s = lax.axis_index("s")

        pltpu.sync_copy(src_hbm.at[s], src_v)
        pltpu.sync_copy(dst_hbm.at[s], dst_v)

        zv = jnp.zeros((16,), jnp.float32)

        def zrow(i, carry):
            for t in range(td):
                zbuf[i, pl.ds(t * 16, 16)] = zv
            return carry
        lax.fori_loop(0, zr, zrow, 0)

        def off_row(delta):
            dv = jnp.full((16,), delta, jnp.int32)

            def step(i, carry):
                for t in range(CH // 16):
                    src_v[i, pl.ds(t * 16, 16)] = (
                        src_v[i, pl.ds(t * 16, 16)] + dv)
                return carry
            lax.fori_loop(0, nch, step, 0)

        off_row(c * n)

        nz_t = jnp.minimum(n - s * rpt, rpt) // zr

        for p in range(npass):
            if p > 0:
                off_row(2 * n)

            for z in range(nzero):
                pltpu.sync_copy(zbuf, acc.at[pl.ds(s * rpt + z * zr, zr)])
            plsc.subcore_barrier()

            def g_start(j, b):
                pltpu.async_copy(xr_hbm.at[src_v.at[j]], gbufs[b], gsems[b])
                pltpu.async_copy(w_hbm.at[s, j], wbufs[b], gsems[b])

            def g_wait(b):
                pltpu.make_async_copy(
                    xr_hbm.at[pl.ds(0, CH)], gbufs[b], gsems[b]).wait()
                pltpu.make_async_copy(
                    w_hbm.at[0, 0], wbufs[b], gsems[b]).wait()

            def s_start(j, b):
                pltpu.async_copy(mbufs[b], acc.at[dst_v.at[j]], ssems[b],
                                 add=True)

            def s_wait(b):
                pltpu.make_async_copy(
                    xr_hbm.at[pl.ds(0, CH)], mbufs[b], ssems[b]).wait()

            def multiply(j, b, bs):
                def egroup(g, carry2):
                    for e in range(16):
                        row = g * 16 + e
                        we = wbufs[b][pl.ds(row * 16, 16)]
                        for t in range(td):
                            mbufs[bs][row, pl.ds(t * 16, 16)] = (
                                gbufs[b][row, pl.ds(t * 16, 16)] * we)
                    return carry2
                lax.fori_loop(0, CH // 16, egroup, 0)

            for b in range(NG - 1):
                g_start(b, b)

            def pipe(i, carry):
                for b in range(NG):
                    j = NG * i + b
                    bs = b % NS

                    @pl.when(j + NG - 1 < nch)
                    def _():
                        g_start(j + NG - 1, (b + NG - 1) % NG)
                    g_wait(b)
                    if b >= NS:
                        s_wait(bs)
                    else:
                        @pl.when(i > 0)
                        def _():
                            s_wait(bs)
                    multiply(j, b, bs)
                    s_start(j, bs)
                return carry
            lax.fori_loop(0, nch // NG, pipe, 0)
            for j in range(NG * (nch // NG), nch):
                g_wait(j % NG)
                s_wait(j % NS)
                multiply(j, j % NG, j % NS)
                s_start(j, j % NS)
            for j in range(nch - NS, nch):
                s_wait(j % NS)
            plsc.subcore_barrier()

            base_out = (2 * p + c) * n + s * rpt

            def wout(z, carry):
                for ob in range(2):
                    @pl.when(2 * z + ob < nz_t)
                    def _():
                        zz = 2 * z + ob

                        @pl.when(zz > 1)
                        def _():
                            pltpu.make_async_copy(
                                out_hbm.at[pl.ds(0, zr)], obufs[ob],
                                wsem).wait()
                        pltpu.sync_copy(
                            acc.at[pl.ds(s * rpt + zz * zr, zr)], obufs[ob])
                        pltpu.async_copy(
                            obufs[ob],
                            out_hbm.at[pl.ds(base_out + zz * zr, zr)], wsem)
                return carry
            lax.fori_loop(0, (nz_t + 1) // 2, wout, 0)

            @pl.when(nz_t > 1)
            def _():
                pltpu.make_async_copy(
                    out_hbm.at[pl.ds(0, zr)], obufs[0], wsem).wait()
            pltpu.make_async_copy(
                out_hbm.at[pl.ds(0, zr)], obufs[1], wsem).wait()

    return k


def _mm_front(x, w_rel, w_root, b2, kq):
    n, kdim = x.shape
    dout = w_rel.shape[0]
    dq = dout // kq
    bm = 2000
    nb = n // bm
    dn = (((1,), (1,)), ((), ()))

    def body(x_ref, wrel_ref, wroot_ref, b_ref, xr_ref, xroot_ref):
        xb = x_ref[...]
        yr = lax.dot_general(xb, wrel_ref[...], dn,
                             preferred_element_type=jnp.float32)
        yo = lax.dot_general(xb, wroot_ref[...], dn,
                             preferred_element_type=jnp.float32) + b_ref[...]
        for q in range(kq):
            xr_ref[q, :, :] = yr[:, q * dq:(q + 1) * dq]
            xroot_ref[q, :, :] = yo[:, q * dq:(q + 1) * dq]

    xr3, xroot3 = pl.pallas_call(
        body,
        grid=(nb,),
        in_specs=[
            pl.BlockSpec((bm, kdim), lambda i: (i, 0)),
            pl.BlockSpec((dout, kdim), lambda i: (0, 0)),
            pl.BlockSpec((dout, kdim), lambda i: (0, 0)),
            pl.BlockSpec((1, dout), lambda i: (0, 0)),
        ],
        out_specs=[
            pl.BlockSpec((kq, bm, dq), lambda i: (0, i, 0)),
            pl.BlockSpec((kq, bm, dq), lambda i: (0, i, 0)),
        ],
        out_shape=[
            jax.ShapeDtypeStruct((kq, n, dq), jnp.float32),
            jax.ShapeDtypeStruct((kq, n, dq), jnp.float32),
        ],
    )(x, w_rel, w_root, b2)
    return xr3.reshape(kq * n, dq), xroot3.reshape(kq * n, dq)


def _bn_relu(pre, s0, s1, g_q, be_q, n):
    mean = s0 * (1.0 / n)
    var = s1 * (1.0 / n) - mean * mean
    scale = g_q * lax.rsqrt(var + EPS)
    shift = be_q - mean * scale
    return jnp.maximum(pre * scale[None, :] + shift[None, :], 0.0)


def _mm_mid(seg, xroot, gammaq, betaq, w_rel, w_root, b2, n, kq_in, kq_out):
    dq = seg.shape[1]
    dout = w_rel.shape[0]
    din = w_rel.shape[1]
    dqn = dout // kq_out
    nb = n // BM
    dn = (((1,), (1,)), ((), ()))

    def body(*refs):
        seg_refs = refs[:kq_in]
        xr_refs = refs[kq_in:2 * kq_in]
        g_ref, be_ref, wrel_ref, wroot_ref, b_ref = refs[2 * kq_in:-3]
        xr_out, xroot_out = refs[-3:-1]
        sums = refs[-1]
        p = pl.program_id(0)
        i = pl.program_id(1)

        @pl.when(p == 0)
        def _():
            @pl.when(i == 0)
            def _():
                sums[...] = jnp.zeros((2 * kq_in, dq), jnp.float32)
            for q in range(kq_in):
                pre = seg_refs[q][...] + xr_refs[q][...]
                sums[pl.ds(2 * q, 1), :] += jnp.sum(pre, axis=0, keepdims=True)
                sums[pl.ds(2 * q + 1, 1), :] += jnp.sum(pre * pre, axis=0,
                                                        keepdims=True)

        @pl.when(p == 1)
        def _():
            sv = sums[...]
            hs = [
                _bn_relu(seg_refs[q][...] + xr_refs[q][...],
                         sv[2 * q], sv[2 * q + 1], g_ref[q], be_ref[q], n)
                for q in range(kq_in)
            ]
            hcat = jnp.concatenate(hs, axis=1)
            yr = lax.dot_general(hcat, wrel_ref[...], dn,
                                 preferred_element_type=jnp.float32)
            yo = lax.dot_general(hcat, wroot_ref[...], dn,
                                 preferred_element_type=jnp.float32) + b_ref[...]
            for q in range(kq_out):
                xr_out[q, :, :] = yr[:, q * dqn:(q + 1) * dqn]
                xroot_out[q, :, :] = yo[:, q * dqn:(q + 1) * dqn]

    def qmap(q):
        return lambda p, i: (q * nb + i, 0)

    in_specs = (
        [pl.BlockSpec((BM, dq), qmap(q)) for q in range(kq_in)]
        + [pl.BlockSpec((BM, dq), qmap(q)) for q in range(kq_in)]
        + [
            pl.BlockSpec((kq_in, dq), lambda p, i: (0, 0)),
            pl.BlockSpec((kq_in, dq), lambda p, i: (0, 0)),
            pl.BlockSpec((dout, din), lambda p, i: (0, 0)),
            pl.BlockSpec((dout, din), lambda p, i: (0, 0)),
            pl.BlockSpec((1, dout), lambda p, i: (0, 0)),
        ]
    )
    xr3, xroot3 = pl.pallas_call(
        body,
        grid=(2, nb),
        in_specs=in_specs,
        out_specs=[
            pl.BlockSpec((kq_out, BM, dqn), lambda p, i: (0, i * p, 0)),
            pl.BlockSpec((kq_out, BM, dqn), lambda p, i: (0, i * p, 0)),
        ],
        out_shape=[
            jax.ShapeDtypeStruct((kq_out, n, dqn), jnp.float32),
            jax.ShapeDtypeStruct((kq_out, n, dqn), jnp.float32),
        ],
        scratch_shapes=[pltpu.VMEM((2 * kq_in, dq), jnp.float32)],
    )(*([seg] * kq_in), *([xroot] * kq_in),
      gammaq, betaq, w_rel, w_root, b2)
    return xr3.reshape(kq_out * n, dqn), xroot3.reshape(kq_out * n, dqn)


def _final(seg, xroot, gammaq, betaq, w_lin, b_lin, n, kq_in):
    dn = (((1,), (1,)), ((), ()))

    def body(s_ref, x_ref, g_ref, be_ref, wl_ref, bl_ref, out_ref):
        hs = []
        for q in range(kq_in):
            pre = s_ref[pl.ds(q * n, n), :] + x_ref[pl.ds(q * n, n), :]
            s0 = jnp.sum(pre, axis=0)
            s1 = jnp.sum(pre * pre, axis=0)
            hs.append(_bn_relu(pre, s0, s1, g_ref[q], be_ref[q], n))
        hcat = jnp.concatenate(hs, axis=1)
        y = lax.dot_general(hcat, wl_ref[...], dn,
                            preferred_element_type=jnp.float32) + bl_ref[0, 0]
        z = jnp.minimum(y, 0.0) - jnp.log(1.0 + jnp.exp(-jnp.abs(y)))
        col0 = lax.broadcasted_iota(jnp.int32, z.shape, 1) == 0
        m = jnp.max(jnp.where(col0, z, -jnp.inf))
        lse = m + jnp.log(jnp.sum(jnp.where(col0, jnp.exp(z - m), 0.0)))
        out_ref[...] = z - lse

    return pl.pallas_call(
        body,
        out_shape=jax.ShapeDtypeStruct((n, 8), jnp.float32),
    )(seg, xroot, gammaq, betaq, w_lin, b_lin)


def kernel(x, edge_index, edge_weight,
           W_rel1, b_rel1, W_root1, gamma1, beta1,
           W_rel2, b_rel2, W_root2, gamma2, beta2,
           W_rel3, b_rel3, W_root3, gamma3, beta3,
           W_lin, b_lin):
    n = x.shape[0]
    e = edge_weight.shape[0]
    per = -(-e // NSUB)
    per_p = -(-per // CH) * CH
    pad = NSUB * per_p - e
    nch = per_p // CH

    src = jnp.concatenate(
        [edge_index[0], jnp.zeros((pad,), jnp.int32)]).reshape(NSUB, nch, CH)
    dst = jnp.concatenate(
        [edge_index[1], jnp.zeros((pad,), jnp.int32)]).reshape(NSUB, nch, CH)
    wgt = jnp.broadcast_to(
        jnp.concatenate(
            [edge_weight, jnp.zeros((pad,), jnp.float32)])[:, None],
        (NSUB * per_p, 16)).reshape(NSUB, nch, CH * 16)

    kq1, kq2, kq3 = 4, 2, 2
    sc1 = _make_sc_seg(n, W_rel1.shape[0] // kq1, nch, kq1 // 2)
    sc2 = _make_sc_seg(n, W_rel2.shape[0] // kq2, nch, kq2 // 2)
    sc3 = _make_sc_seg(n, W_rel3.shape[0] // kq3, nch, kq3 // 2)

    xr, xroot = _mm_front(x, W_rel1, W_root1, b_rel1.reshape(1, -1), kq1)
    seg = sc1(xr, src, dst, wgt)
    xr, xroot = _mm_mid(seg, xroot, gamma1.reshape(kq1, -1),
                        beta1.reshape(kq1, -1), W_rel2, W_root2,
                        b_rel2.reshape(1, -1), n, kq1, kq2)
    seg = sc2(xr, src, dst, wgt)
    xr, xroot = _mm_mid(seg, xroot, gamma2.reshape(kq2, -1),
                        beta2.reshape(kq2, -1), W_rel3, W_root3,
                        b_rel3.reshape(1, -1), n, kq2, kq3)
    seg = sc3(xr, src, dst, wgt)
    w_lin8 = jnp.pad(W_lin, ((0, 7), (0, 0)))
    y8 = _final(seg, xroot, gamma3.reshape(kq3, -1),
                beta3.reshape(kq3, -1), w_lin8, b_lin.reshape(1, 1), n, kq3)
    return y8[:, :1]


# speedup vs baseline: 1.1358x; 1.1358x over previous
import functools

import jax
import jax.numpy as jnp
from jax import lax
from jax.experimental import pallas as pl
from jax.experimental.pallas import tpu as pltpu
from jax.experimental.pallas import tpu_sc as plsc

NSUB = 16
NG = 2
NS = 2
CH = 128
BM = 1000
EPS = 1e-5


def _make_sc_seg(n, dq, nch, npass):
    td = dq // 16
    assert nch % 4 == 3 and nch >= 7
    zr = 80
    rpt = -(-(-(-n // NSUB)) // zr) * zr
    nzero = rpt // zr
    kq = 2 * npass
    mesh = plsc.VectorSubcoreMesh(core_axis_name="c", subcore_axis_name="s",
                                  num_cores=2, num_subcores=NSUB)

    @functools.partial(
        pl.kernel,
        out_type=jax.ShapeDtypeStruct((kq * n, dq), jnp.float32),
        mesh=mesh,
        scratch_types=[
            pltpu.VMEM((nch, CH), jnp.int32),
            pltpu.VMEM((nch, CH), jnp.int32),
            pltpu.VMEM((nch * CH,), jnp.float32),
            [pltpu.VMEM((CH, dq), jnp.float32)] * NG,
            [pltpu.VMEM((CH, dq), jnp.float32)] * NS,
            pltpu.VMEM((zr, dq), jnp.float32),
            [pltpu.VMEM((zr, dq), jnp.float32)] * 2,
            pltpu.VMEM_SHARED((NSUB * rpt, dq), jnp.float32),
            [pltpu.SemaphoreType.DMA] * NG,
            [pltpu.SemaphoreType.DMA] * NS,
            pltpu.SemaphoreType.DMA,
        ],
        compiler_params=pltpu.CompilerParams(use_tc_tiling_on_sc=False,
                                             skip_device_barrier=True),
    )
    def k(xr_hbm, src_hbm, dst_hbm, w_hbm, out_hbm,
          src_v, dst_v, w_v, gbufs, mbufs, zbuf, obufs, acc,
          gsems, ssems, wsem):
        c = lax.axis_index("c")
        s = lax.axis_index("s")

        pltpu.sync_copy(src_hbm.at[s], src_v)
        pltpu.sync_copy(dst_hbm.at[s], dst_v)
        pltpu.sync_copy(w_hbm.at[s], w_v)

        zv = jnp.zeros((16,), jnp.float32)

        def zrow(i, carry):
            for t in range(td):
                zbuf[i, pl.ds(t * 16, 16)] = zv
            return carry
        lax.fori_loop(0, zr, zrow, 0)

        def off_row(delta):
            dv = jnp.full((16,), delta, jnp.int32)

            def step(i, carry):
                for t in range(CH // 16):
                    src_v[i, pl.ds(t * 16, 16)] = (
                        src_v[i, pl.ds(t * 16, 16)] + dv)
                return carry
            lax.fori_loop(0, nch, step, 0)

        off_row(c * n)

        nz_t = jnp.minimum(n - s * rpt, rpt) // zr

        for p in range(npass):
            if p > 0:
                off_row(2 * n)

            for z in range(nzero):
                pltpu.async_copy(zbuf, acc.at[pl.ds(s * rpt + z * zr, zr)],
                                 wsem)
            for z in range(nzero):
                pltpu.make_async_copy(
                    out_hbm.at[pl.ds(0, zr)], zbuf, wsem).wait()
            plsc.subcore_barrier()

            def g_start(j, b):
                pltpu.async_copy(xr_hbm.at[src_v.at[j]], gbufs[b], gsems[b])

            def g_wait(b):
                pltpu.make_async_copy(
                    xr_hbm.at[pl.ds(0, CH)], gbufs[b], gsems[b]).wait()

            def s_start(j, b):
                pltpu.async_copy(mbufs[b], acc.at[dst_v.at[j]], ssems[b],
                                 add=True)

            def s_wait(b):
                pltpu.make_async_copy(
                    xr_hbm.at[pl.ds(0, CH)], mbufs[b], ssems[b]).wait()

            def multiply(j, b, bs):
                def egroup(g, carry2):
                    wv16 = w_v[pl.ds(j * CH + g * 16, 16)]
                    for e in range(16):
                        we = jnp.full((16,), wv16[e], jnp.float32)
                        row = g * 16 + e
                        for t in range(td):
                            mbufs[bs][row, pl.ds(t * 16, 16)] = (
                                gbufs[b][row, pl.ds(t * 16, 16)] * we)
                    return carry2
                lax.fori_loop(0, CH // 16, egroup, 0)

            for b in range(NG - 1):
                g_start(b, b)

            def pipe(i, carry):
                for b in range(NG):
                    j = NG * i + b
                    bs = b % NS

                    @pl.when(j + NG - 1 < nch)
                    def _():
                        g_start(j + NG - 1, (b + NG - 1) % NG)
                    g_wait(b)
                    if b >= NS:
                        s_wait(bs)
                    else:
                        @pl.when(i > 0)
                        def _():
                            s_wait(bs)
                    multiply(j, b, bs)
                    s_start(j, bs)
                return carry
            lax.fori_loop(0, nch // NG, pipe, 0)
            for j in range(NG * (nch // NG), nch):
                g_wait(j % NG)
                s_wait(j % NS)
                multiply(j, j % NG, j % NS)
                s_start(j, j % NS)
            for j in range(nch - NS, nch):
                s_wait(j % NS)
            plsc.subcore_barrier()

            base_out = (2 * p + c) * n + s * rpt

            def wout(z, carry):
                for ob in range(2):
                    @pl.when(2 * z + ob < nz_t)
                    def _():
                        zz = 2 * z + ob

                        @pl.when(zz > 1)
                        def _():
                            pltpu.make_async_copy(
                                out_hbm.at[pl.ds(0, zr)], obufs[ob],
                                wsem).wait()
                        pltpu.sync_copy(
                            acc.at[pl.ds(s * rpt + zz * zr, zr)], obufs[ob])
                        pltpu.async_copy(
                            obufs[ob],
                            out_hbm.at[pl.ds(base_out + zz * zr, zr)], wsem)
                return carry
            lax.fori_loop(0, (nz_t + 1) // 2, wout, 0)

            @pl.when(nz_t > 1)
            def _():
                pltpu.make_async_copy(
                    out_hbm.at[pl.ds(0, zr)], obufs[0], wsem).wait()
            pltpu.make_async_copy(
                out_hbm.at[pl.ds(0, zr)], obufs[1], wsem).wait()

    return k


def _mm_front(x, w_rel, w_root, b2, kq):
    n, kdim = x.shape
    dout = w_rel.shape[0]
    dq = dout // kq
    bm = 2000
    nb = n // bm
    dn = (((1,), (1,)), ((), ()))

    def body(x_ref, wrel_ref, wroot_ref, b_ref, xr_ref, xroot_ref):
        xb = x_ref[...]
        yr = lax.dot_general(xb, wrel_ref[...], dn,
                             preferred_element_type=jnp.float32)
        yo = lax.dot_general(xb, wroot_ref[...], dn,
                             preferred_element_type=jnp.float32) + b_ref[...]
        for q in range(kq):
            xr_ref[q, :, :] = yr[:, q * dq:(q + 1) * dq]
            xroot_ref[q, :, :] = yo[:, q * dq:(q + 1) * dq]

    xr3, xroot3 = pl.pallas_call(
        body,
        grid=(nb,),
        in_specs=[
            pl.BlockSpec((bm, kdim), lambda i: (i, 0)),
            pl.BlockSpec((dout, kdim), lambda i: (0, 0)),
            pl.BlockSpec((dout, kdim), lambda i: (0, 0)),
            pl.BlockSpec((1, dout), lambda i: (0, 0)),
        ],
        out_specs=[
            pl.BlockSpec((kq, bm, dq), lambda i: (0, i, 0)),
            pl.BlockSpec((kq, bm, dq), lambda i: (0, i, 0)),
        ],
        out_shape=[
            jax.ShapeDtypeStruct((kq, n, dq), jnp.float32),
            jax.ShapeDtypeStruct((kq, n, dq), jnp.float32),
        ],
    )(x, w_rel, w_root, b2)
    return xr3.reshape(kq * n, dq), xroot3.reshape(kq * n, dq)


def _bn_relu(pre, s0, s1, g_q, be_q, n):
    mean = s0 * (1.0 / n)
    var = s1 * (1.0 / n) - mean * mean
    scale = g_q * lax.rsqrt(var + EPS)
    shift = be_q - mean * scale
    return jnp.maximum(pre * scale[None, :] + shift[None, :], 0.0)


def _mm_mid(seg, xroot, gammaq, betaq, w_rel, w_root, b2, n, kq_in, kq_out):
    dq = seg.shape[1]
    dout = w_rel.shape[0]
    din = w_rel.shape[1]
    dqn = dout // kq_out
    nb = n // BM
    dn = (((1,), (1,)), ((), ()))

    def body(*refs):
        seg_refs = refs[:kq_in]
        xr_refs = refs[kq_in:2 * kq_in]
        g_ref, be_ref, wrel_ref, wroot_ref, b_ref = refs[2 * kq_in:-3]
        xr_out, xroot_out = refs[-3:-1]
        sums = refs[-1]
        p = pl.program_id(0)
        i = pl.program_id(1)

        @pl.when(p == 0)
        def _():
            @pl.when(i == 0)
            def _():
                sums[...] = jnp.zeros((2 * kq_in, dq), jnp.float32)
            for q in range(kq_in):
                pre = seg_refs[q][...] + xr_refs[q][...]
                sums[pl.ds(2 * q, 1), :] += jnp.sum(pre, axis=0, keepdims=True)
                sums[pl.ds(2 * q + 1, 1), :] += jnp.sum(pre * pre, axis=0,
                                                        keepdims=True)

        @pl.when(p == 1)
        def _():
            sv = sums[...]
            hs = [
                _bn_relu(seg_refs[q][...] + xr_refs[q][...],
                         sv[2 * q], sv[2 * q + 1], g_ref[q], be_ref[q], n)
                for q in range(kq_in)
            ]
            hcat = jnp.concatenate(hs, axis=1)
            yr = lax.dot_general(hcat, wrel_ref[...], dn,
                                 preferred_element_type=jnp.float32)
            yo = lax.dot_general(hcat, wroot_ref[...], dn,
                                 preferred_element_type=jnp.float32) + b_ref[...]
            for q in range(kq_out):
                xr_out[q, :, :] = yr[:, q * dqn:(q + 1) * dqn]
                xroot_out[q, :, :] = yo[:, q * dqn:(q + 1) * dqn]

    def qmap(q):
        return lambda p, i: (q * nb + i, 0)

    in_specs = (
        [pl.BlockSpec((BM, dq), qmap(q)) for q in range(kq_in)]
        + [pl.BlockSpec((BM, dq), qmap(q)) for q in range(kq_in)]
        + [
            pl.BlockSpec((kq_in, dq), lambda p, i: (0, 0)),
            pl.BlockSpec((kq_in, dq), lambda p, i: (0, 0)),
            pl.BlockSpec((dout, din), lambda p, i: (0, 0)),
            pl.BlockSpec((dout, din), lambda p, i: (0, 0)),
            pl.BlockSpec((1, dout), lambda p, i: (0, 0)),
        ]
    )
    xr3, xroot3 = pl.pallas_call(
        body,
        grid=(2, nb),
        in_specs=in_specs,
        out_specs=[
            pl.BlockSpec((kq_out, BM, dqn), lambda p, i: (0, i * p, 0)),
            pl.BlockSpec((kq_out, BM, dqn), lambda p, i: (0, i * p, 0)),
        ],
        out_shape=[
            jax.ShapeDtypeStruct((kq_out, n, dqn), jnp.float32),
            jax.ShapeDtypeStruct((kq_out, n, dqn), jnp.float32),
        ],
        scratch_shapes=[pltpu.VMEM((2 * kq_in, dq), jnp.float32)],
    )(*([seg] * kq_in), *([xroot] * kq_in),
      gammaq, betaq, w_rel, w_root, b2)
    return xr3.reshape(kq_out * n, dqn), xroot3.reshape(kq_out * n, dqn)


def _final(seg, xroot, gammaq, betaq, w_lin, b_lin, n, kq_in):
    dn = (((1,), (1,)), ((), ()))

    def body(s_ref, x_ref, g_ref, be_ref, wl_ref, bl_ref, out_ref):
        hs = []
        for q in range(kq_in):
            pre = s_ref[pl.ds(q * n, n), :] + x_ref[pl.ds(q * n, n), :]
            s0 = jnp.sum(pre, axis=0)
            s1 = jnp.sum(pre * pre, axis=0)
            hs.append(_bn_relu(pre, s0, s1, g_ref[q], be_ref[q], n))
        hcat = jnp.concatenate(hs, axis=1)
        y = lax.dot_general(hcat, wl_ref[...], dn,
                            preferred_element_type=jnp.float32) + bl_ref[0, 0]
        z = jnp.minimum(y, 0.0) - jnp.log(1.0 + jnp.exp(-jnp.abs(y)))
        col0 = lax.broadcasted_iota(jnp.int32, z.shape, 1) == 0
        m = jnp.max(jnp.where(col0, z, -jnp.inf))
        lse = m + jnp.log(jnp.sum(jnp.where(col0, jnp.exp(z - m), 0.0)))
        out_ref[...] = z - lse

    return pl.pallas_call(
        body,
        out_shape=jax.ShapeDtypeStruct((n, 8), jnp.float32),
    )(seg, xroot, gammaq, betaq, w_lin, b_lin)


def kernel(x, edge_index, edge_weight,
           W_rel1, b_rel1, W_root1, gamma1, beta1,
           W_rel2, b_rel2, W_root2, gamma2, beta2,
           W_rel3, b_rel3, W_root3, gamma3, beta3,
           W_lin, b_lin):
    n = x.shape[0]
    e = edge_weight.shape[0]
    per = -(-e // NSUB)
    per_p = -(-per // CH) * CH
    pad = NSUB * per_p - e
    nch = per_p // CH

    src = jnp.concatenate(
        [edge_index[0], jnp.zeros((pad,), jnp.int32)]).reshape(NSUB, nch, CH)
    dst = jnp.concatenate(
        [edge_index[1], jnp.zeros((pad,), jnp.int32)]).reshape(NSUB, nch, CH)
    wgt = jnp.concatenate(
        [edge_weight, jnp.zeros((pad,), jnp.float32)]).reshape(NSUB, nch * CH)

    kq1, kq2, kq3 = 4, 2, 2
    sc1 = _make_sc_seg(n, W_rel1.shape[0] // kq1, nch, kq1 // 2)
    sc2 = _make_sc_seg(n, W_rel2.shape[0] // kq2, nch, kq2 // 2)
    sc3 = _make_sc_seg(n, W_rel3.shape[0] // kq3, nch, kq3 // 2)

    xr, xroot = _mm_front(x, W_rel1, W_root1, b_rel1.reshape(1, -1), kq1)
    seg = sc1(xr, src, dst, wgt)
    xr, xroot = _mm_mid(seg, xroot, gamma1.reshape(kq1, -1),
                        beta1.reshape(kq1, -1), W_rel2, W_root2,
                        b_rel2.reshape(1, -1), n, kq1, kq2)
    seg = sc2(xr, src, dst, wgt)
    xr, xroot = _mm_mid(seg, xroot, gamma2.reshape(kq2, -1),
                        beta2.reshape(kq2, -1), W_rel3, W_root3,
                        b_rel3.reshape(1, -1), n, kq2, kq3)
    seg = sc3(xr, src, dst, wgt)
    w_lin8 = jnp.pad(W_lin, ((0, 7), (0, 0)))
    y8 = _final(seg, xroot, gamma3.reshape(kq3, -1),
                beta3.reshape(kq3, -1), w_lin8, b_lin.reshape(1, 1), n, kq3)
    return y8[:, :1]
